# baseline (speedup 1.0000x reference)
import functools

import jax
import jax.numpy as jnp
from jax import lax
from jax.experimental import pallas as pl
from jax.experimental.pallas import tpu as pltpu
from jax.experimental.pallas import tpu_sc as plsc

_CHUNK = 16


def _make_gather(B, V, D, num_cores, num_subcores):
    NW = num_cores * num_subcores
    b_per_w = B // NW
    mesh = plsc.VectorSubcoreMesh(core_axis_name="c", subcore_axis_name="s")

    @functools.partial(
        pl.kernel,
        mesh=mesh,
        out_type=jax.ShapeDtypeStruct((B, D), jnp.float32),
        scratch_types=[
            pltpu.VMEM((b_per_w,), jnp.int32),
            pltpu.SemaphoreType.DMA,
        ],
    )
    def gather_kernel(idx_hbm, table_hbm, out_hbm, idx_v, sem):
        wid = lax.axis_index("s") * num_cores + lax.axis_index("c")
        base = wid * b_per_w
        pltpu.sync_copy(idx_hbm.at[pl.ds(base, b_per_w)], idx_v)

        def chunk_body(g, carry):
            off = g * _CHUNK
            idx_vec = idx_v[pl.ds(off, _CHUNK)]
            for j in range(_CHUNK):
                s = idx_vec[j]
                pltpu.async_copy(
                    table_hbm.at[pl.ds(s, 1)],
                    out_hbm.at[pl.ds(base + off + j, 1)],
                    sem,
                )
            for j in range(_CHUNK):
                pltpu.make_async_copy(
                    table_hbm.at[pl.ds(0, 1)],
                    out_hbm.at[pl.ds(base, 1)],
                    sem,
                ).wait()
            return carry

        lax.fori_loop(0, b_per_w // _CHUNK, chunk_body, 0)

    return gather_kernel


def kernel(index, train_score):
    index = index.astype(jnp.int32)
    B = index.shape[0]
    V, D = train_score.shape
    info = plsc.get_sparse_core_info()
    fn = _make_gather(B, V, D, info.num_cores, info.num_subcores)
    return fn(index, train_score)


# speedup vs baseline: 1.9879x; 1.9879x over previous
import functools

import jax
import jax.numpy as jnp
from jax import lax
from jax.experimental import pallas as pl
from jax.experimental.pallas import tpu as pltpu
from jax.experimental.pallas import tpu_sc as plsc

_ROWS_PER_STEP = 800
_GCHUNK = 256


def _widen_body(table_ref, wide_ref):
    block = table_ref[...]
    wide_ref[:, 0:64] = block
    wide_ref[:, 64:128] = block


def _widen(table, V, D, W):
    grid = V // _ROWS_PER_STEP
    return pl.pallas_call(
        _widen_body,
        grid=(grid,),
        in_specs=[pl.BlockSpec((_ROWS_PER_STEP, D), lambda i: (i, 0))],
        out_specs=pl.BlockSpec((_ROWS_PER_STEP, W), lambda i: (i, 0)),
        out_shape=jax.ShapeDtypeStruct((V, W), jnp.float32),
    )(table)


def _make_gather(B, V, W, num_cores, num_subcores):
    NW = num_cores * num_subcores
    b_per_w = B // NW
    n_chunks = b_per_w // _GCHUNK
    mesh = plsc.VectorSubcoreMesh(core_axis_name="c", subcore_axis_name="s")

    @functools.partial(
        pl.kernel,
        mesh=mesh,
        out_type=jax.ShapeDtypeStruct((B, W), jnp.float32),
        scratch_types=[
            pltpu.VMEM((b_per_w,), jnp.int32),
            pltpu.VMEM((_GCHUNK, W), jnp.float32),
            pltpu.SemaphoreType.DMA,
        ],
    )
    def gather_kernel(idx_hbm, wide_hbm, out_hbm, idx_v, rows_v, sem):
        wid = lax.axis_index("s") * num_cores + lax.axis_index("c")
        base = pl.multiple_of(wid * b_per_w, 8)
        pltpu.sync_copy(idx_hbm.at[pl.ds(base, b_per_w)], idx_v)

        def chunk_body(g, carry):
            off = pl.multiple_of(g * _GCHUNK, 8)
            pltpu.async_copy(
                wide_hbm.at[idx_v.at[pl.ds(off, _GCHUNK)]], rows_v, sem
            ).wait()
            pltpu.sync_copy(rows_v, out_hbm.at[pl.ds(base + off, _GCHUNK)])
            return carry

        lax.fori_loop(0, n_chunks, chunk_body, 0)

    return gather_kernel


def kernel(index, train_score):
    index = index.astype(jnp.int32)
    B = index.shape[0]
    V, D = train_score.shape
    W = 2 * D
    info = plsc.get_sparse_core_info()
    wide = _widen(train_score, V, D, W)
    gather = _make_gather(B, V, W, info.num_cores, info.num_subcores)
    out128 = gather(index, wide)
    return lax.slice(out128, (0, 0), (B, D))


# speedup vs baseline: 3.6008x; 1.8113x over previous
import functools

import jax
import jax.numpy as jnp
from jax import lax
from jax.experimental import pallas as pl
from jax.experimental.pallas import tpu as pltpu
from jax.experimental.pallas import tpu_sc as plsc

_ROWS_PER_STEP = 800
_GCHUNK = 256


def _widen_body(table_ref, wide_ref):
    block = table_ref[...]
    wide_ref[:, 0:64] = block
    wide_ref[:, 64:128] = block


def _widen(table, V, D, W):
    grid = V // _ROWS_PER_STEP
    return pl.pallas_call(
        _widen_body,
        grid=(grid,),
        in_specs=[pl.BlockSpec((_ROWS_PER_STEP, D), lambda i: (i, 0))],
        out_specs=pl.BlockSpec((_ROWS_PER_STEP, W), lambda i: (i, 0)),
        out_shape=jax.ShapeDtypeStruct((V, W), jnp.float32),
    )(table)


def _make_gather(B, V, W, num_cores, num_subcores):
    NW = num_cores * num_subcores
    b_per_w = B // NW
    n_chunks = b_per_w // _GCHUNK
    mesh = plsc.VectorSubcoreMesh(core_axis_name="c", subcore_axis_name="s")

    @functools.partial(
        pl.kernel,
        mesh=mesh,
        out_type=jax.ShapeDtypeStruct((B, W), jnp.float32),
        scratch_types=[
            pltpu.VMEM((b_per_w,), jnp.int32),
            pltpu.VMEM((_GCHUNK, W), jnp.float32),
            pltpu.SemaphoreType.DMA,
        ],
    )
    def gather_kernel(idx_hbm, wide_hbm, out_hbm, idx_v, rows_v, sem):
        wid = lax.axis_index("s") * num_cores + lax.axis_index("c")
        base = pl.multiple_of(wid * b_per_w, 8)
        pltpu.sync_copy(idx_hbm.at[pl.ds(base, b_per_w)], idx_v)

        def chunk_body(g, carry):
            off = pl.multiple_of(g * _GCHUNK, 8)
            pltpu.async_copy(
                wide_hbm.at[idx_v.at[pl.ds(off, _GCHUNK)]], rows_v, sem
            ).wait()
            pltpu.sync_copy(rows_v, out_hbm.at[pl.ds(base + off, _GCHUNK)])
            return carry

        lax.fori_loop(0, n_chunks, chunk_body, 0)

    return gather_kernel


def kernel(index, train_score):
    index = index.astype(jnp.int32)
    B = index.shape[0]
    V, D = train_score.shape
    W = 2 * D
    info = plsc.get_sparse_core_info()
    wide = jnp.pad(train_score, ((0, 0), (0, W - D)))
    gather = _make_gather(B, V, W, info.num_cores, info.num_subcores)
    out128 = gather(index, wide)
    return lax.slice(out128, (0, 0), (B, D))
